# baseline (device time: 23056 ns/iter reference)
import jax
import jax.numpy as jnp
from jax import lax
from jax.experimental import pallas as pl
from jax.experimental.pallas import tpu as pltpu

N_DEV = 8
C = 48
_J_ORDER = (6, 7, 5, 2, 4, 3, 1)


def _chunk_slot(q):
    qq = q % 4
    qz = q // 4
    qy = qq // 2
    qx = (qq % 2) ^ qy
    return qx * 4 + qy * 2 + qz


def kernel(table, idx):
    rows_per, d = table.shape
    n = idx.shape[0]
    hc = n // N_DEV

    my = lax.axis_index("i")
    t_my = _chunk_slot(my)

    idxc = idx.reshape(N_DEV, hc)
    kar = jnp.arange(C)
    iar = jnp.arange(hc)

    local_me = idxc - my * rows_per
    chunk_idx = idxc[t_my]
    local_r = chunk_idx[None, :] - (jnp.arange(N_DEV) * rows_per)[:, None]
    loc2 = jnp.concatenate([local_me, local_r], axis=0)
    owned2 = (loc2 >= 0) & (loc2 < rows_per)
    rank2 = jnp.cumsum(owned2, axis=-1) - 1
    sel2 = owned2[:, None, :] & (rank2[:, None, :] == kar[None, :, None])
    mult = jnp.concatenate(
        [jnp.where(owned2[:8], loc2[:8], 0),
         jnp.broadcast_to(iar[None, :], (N_DEV, hc))],
        axis=0,
    )
    comb = (sel2 * mult[:, None, :]).sum(-1)
    valid2 = kar[None, :] < owned2.sum(-1)[:, None]

    b = (
        table[comb[:8].reshape(-1)].astype(jnp.bfloat16)
        * valid2[:8].reshape(-1)[:, None].astype(jnp.bfloat16)
    )
    posm = comb[8:].astype(jnp.int32)
    validm = valid2[8:].astype(jnp.bfloat16)

    sources = lax.bitwise_xor(my, jnp.array((0,) + _J_ORDER))
    posf = posm[sources].reshape(1, N_DEV * C)
    validf = validm[sources].reshape(1, N_DEV * C)

    def body(b_ref, posf_ref, validf_ref, out_ref, bigb_ref, s1, r1, s2, r2):
        p = lax.axis_index("i")

        barrier_sem = pltpu.get_barrier_semaphore()
        for j in range(1, N_DEV):
            pl.semaphore_signal(
                barrier_sem, inc=1,
                device_id=(lax.bitwise_xor(p, j),),
                device_id_type=pl.DeviceIdType.MESH,
            )
        pl.semaphore_wait(barrier_sem, N_DEV - 1)

        sends1 = {}
        for m, j in enumerate(_J_ORDER):
            q = lax.bitwise_xor(p, j)
            rdma = pltpu.make_async_remote_copy(
                src_ref=b_ref.at[pl.ds(_chunk_slot(q) * C, C)],
                dst_ref=bigb_ref.at[pl.ds((1 + m) * C, C)],
                send_sem=s1.at[j - 1],
                recv_sem=r1.at[j - 1],
                device_id=(q,),
                device_id_type=pl.DeviceIdType.MESH,
            )
            rdma.start()
            sends1[j] = rdma

        tp = _chunk_slot(p)
        bigb_ref[pl.ds(0, C)] = b_ref[pl.ds(tp * C, C)]
        pall = (
            lax.broadcasted_iota(jnp.int32, (hc, N_DEV * C), 0)
            == posf_ref[...]
        ).astype(jnp.bfloat16) * validf_ref[...]

        for j in _J_ORDER:
            sends1[j].wait_recv()

        acc = lax.dot_general(
            pall,
            bigb_ref[...],
            (((1,), (0,)), ((), ())),
            preferred_element_type=jnp.float32,
        )
        my_off = tp * hc
        out_ref[pl.ds(my_off, hc)] = acc.astype(jnp.bfloat16)

        sends2 = {}
        for j in _J_ORDER:
            q = lax.bitwise_xor(p, j)
            rdma = pltpu.make_async_remote_copy(
                src_ref=out_ref.at[pl.ds(my_off, hc)],
                dst_ref=out_ref.at[pl.ds(my_off, hc)],
                send_sem=s2.at[j - 1],
                recv_sem=r2.at[j - 1],
                device_id=(q,),
                device_id_type=pl.DeviceIdType.MESH,
            )
            rdma.start()
            sends2[j] = rdma
        for j in _J_ORDER:
            sends2[j].wait_recv()
        for j in _J_ORDER:
            sends1[j].wait_send()
            sends2[j].wait_send()

    out = pl.pallas_call(
        body,
        out_shape=jax.ShapeDtypeStruct((n, d), jnp.bfloat16),
        in_specs=[
            pl.BlockSpec(memory_space=pltpu.VMEM),
            pl.BlockSpec(memory_space=pltpu.VMEM),
            pl.BlockSpec(memory_space=pltpu.VMEM),
        ],
        out_specs=pl.BlockSpec(memory_space=pltpu.VMEM),
        scratch_shapes=[
            pltpu.VMEM((N_DEV * C, d), jnp.bfloat16),
            pltpu.SemaphoreType.DMA((N_DEV - 1,)),
            pltpu.SemaphoreType.DMA((N_DEV - 1,)),
            pltpu.SemaphoreType.DMA((N_DEV - 1,)),
            pltpu.SemaphoreType.DMA((N_DEV - 1,)),
        ],
        compiler_params=pltpu.CompilerParams(collective_id=0),
    )(b, posf, validf)
    return out


# device time: 21659 ns/iter; 1.0645x vs baseline; 1.0645x over previous
import jax
import jax.numpy as jnp
from jax import lax
from jax.experimental import pallas as pl
from jax.experimental.pallas import tpu as pltpu

N_DEV = 8
C = 40
_J_ORDER = (6, 7, 5, 2, 4, 3, 1)


def _chunk_slot(q):
    qq = q % 4
    qz = q // 4
    qy = qq // 2
    qx = (qq % 2) ^ qy
    return qx * 4 + qy * 2 + qz


def kernel(table, idx):
    rows_per, d = table.shape
    n = idx.shape[0]
    hc = n // N_DEV

    my = lax.axis_index("i")
    t_my = _chunk_slot(my)

    idxc = idx.reshape(N_DEV, hc)
    kar = jnp.arange(C)
    iar = jnp.arange(hc)

    local_me = idxc - my * rows_per
    chunk_idx = idxc[t_my]
    local_r = chunk_idx[None, :] - (jnp.arange(N_DEV) * rows_per)[:, None]
    loc2 = jnp.concatenate([local_me, local_r], axis=0)
    owned2 = (loc2 >= 0) & (loc2 < rows_per)
    rank2 = jnp.cumsum(owned2, axis=-1) - 1
    sel2 = owned2[:, None, :] & (rank2[:, None, :] == kar[None, :, None])
    mult = jnp.concatenate(
        [jnp.where(owned2[:8], loc2[:8], 0),
         jnp.broadcast_to(iar[None, :], (N_DEV, hc))],
        axis=0,
    )
    comb = (sel2 * mult[:, None, :]).sum(-1)
    valid2 = kar[None, :] < owned2.sum(-1)[:, None]

    b = (
        table[comb[:8].reshape(-1)].astype(jnp.bfloat16)
        * valid2[:8].reshape(-1)[:, None].astype(jnp.bfloat16)
    )
    posm = comb[8:].astype(jnp.int32)
    validm = valid2[8:].astype(jnp.bfloat16)

    def placement(posall, valall, s):
        m = lax.broadcasted_iota(jnp.int32, (N_DEV, C), 0) == s
        prow = jnp.sum(posall * m, axis=0, keepdims=True)
        vrow = jnp.sum(valall * m.astype(jnp.bfloat16), axis=0, keepdims=True)
        eq = lax.broadcasted_iota(jnp.int32, (hc, C), 0) == prow
        return eq.astype(jnp.bfloat16) * vrow

    def body(b_ref, pos_ref, val_ref, out_ref, rbuf_ref, s1, r1, s2, r2):
        p = lax.axis_index("i")

        barrier_sem = pltpu.get_barrier_semaphore()
        for j in range(1, N_DEV):
            pl.semaphore_signal(
                barrier_sem, inc=1,
                device_id=(lax.bitwise_xor(p, j),),
                device_id_type=pl.DeviceIdType.MESH,
            )
        pl.semaphore_wait(barrier_sem, N_DEV - 1)

        sends1 = {}
        for j in _J_ORDER:
            q = lax.bitwise_xor(p, j)
            rdma = pltpu.make_async_remote_copy(
                src_ref=b_ref.at[pl.ds(_chunk_slot(q) * C, C)],
                dst_ref=rbuf_ref.at[j - 1],
                send_sem=s1.at[j - 1],
                recv_sem=r1.at[j - 1],
                device_id=(q,),
                device_id_type=pl.DeviceIdType.MESH,
            )
            rdma.start()
            sends1[j] = rdma

        tp = _chunk_slot(p)
        posall = pos_ref[...]
        valall = val_ref[...]
        acc = lax.dot_general(
            placement(posall, valall, p),
            b_ref[pl.ds(tp * C, C)],
            (((1,), (0,)), ((), ())),
            preferred_element_type=jnp.float32,
        )
        for j in _J_ORDER:
            sends1[j].wait_recv()
            s = lax.bitwise_xor(p, j)
            acc = acc + lax.dot_general(
                placement(posall, valall, s),
                rbuf_ref[j - 1],
                (((1,), (0,)), ((), ())),
                preferred_element_type=jnp.float32,
            )
        my_off = tp * hc
        out_ref[pl.ds(my_off, hc)] = acc.astype(jnp.bfloat16)

        sends2 = {}
        for j in _J_ORDER:
            q = lax.bitwise_xor(p, j)
            rdma = pltpu.make_async_remote_copy(
                src_ref=out_ref.at[pl.ds(my_off, hc)],
                dst_ref=out_ref.at[pl.ds(my_off, hc)],
                send_sem=s2.at[j - 1],
                recv_sem=r2.at[j - 1],
                device_id=(q,),
                device_id_type=pl.DeviceIdType.MESH,
            )
            rdma.start()
            sends2[j] = rdma
        for j in _J_ORDER:
            sends2[j].wait_recv()
        for j in _J_ORDER:
            sends1[j].wait_send()
            sends2[j].wait_send()

    out = pl.pallas_call(
        body,
        out_shape=jax.ShapeDtypeStruct((n, d), jnp.bfloat16),
        in_specs=[
            pl.BlockSpec(memory_space=pltpu.VMEM),
            pl.BlockSpec(memory_space=pltpu.VMEM),
            pl.BlockSpec(memory_space=pltpu.VMEM),
        ],
        out_specs=pl.BlockSpec(memory_space=pltpu.VMEM),
        scratch_shapes=[
            pltpu.VMEM((N_DEV - 1, C, d), jnp.bfloat16),
            pltpu.SemaphoreType.DMA((N_DEV - 1,)),
            pltpu.SemaphoreType.DMA((N_DEV - 1,)),
            pltpu.SemaphoreType.DMA((N_DEV - 1,)),
            pltpu.SemaphoreType.DMA((N_DEV - 1,)),
        ],
        compiler_params=pltpu.CompilerParams(collective_id=0),
    )(b, posm, validm)
    return out


# device time: 21601 ns/iter; 1.0674x vs baseline; 1.0027x over previous
import jax
import jax.numpy as jnp
from jax import lax
from jax.experimental import pallas as pl
from jax.experimental.pallas import tpu as pltpu

N_DEV = 8
C = 40
_J_ORDER = (6, 7, 5, 2, 4, 3, 1)


def _chunk_slot(q):
    qq = q % 4
    qz = q // 4
    qy = qq // 2
    qx = (qq % 2) ^ qy
    return qx * 4 + qy * 2 + qz


def kernel(table, idx):
    rows_per, d = table.shape
    n = idx.shape[0]
    hc = n // N_DEV

    my = lax.axis_index("i")
    t_my = _chunk_slot(my)

    idxc = idx.reshape(N_DEV, hc)
    kar = jnp.arange(C)
    iar = jnp.arange(hc)

    local_me = idxc - my * rows_per
    chunk_idx = idxc[t_my]
    local_r = chunk_idx[None, :] - (jnp.arange(N_DEV) * rows_per)[:, None]
    loc2 = jnp.concatenate([local_me, local_r], axis=0)
    owned2 = (loc2 >= 0) & (loc2 < rows_per)
    rank2 = jnp.cumsum(owned2, axis=-1) - 1
    sel2 = owned2[:, None, :] & (rank2[:, None, :] == kar[None, :, None])
    mult = jnp.concatenate(
        [jnp.where(owned2[:8], loc2[:8], 0),
         jnp.broadcast_to(iar[None, :], (N_DEV, hc))],
        axis=0,
    )
    comb = (sel2 * mult[:, None, :]).sum(-1)
    valid2 = kar[None, :] < owned2.sum(-1)[:, None]

    b = (
        table[comb[:8].reshape(-1)].astype(jnp.bfloat16)
        * valid2[:8].reshape(-1)[:, None].astype(jnp.bfloat16)
    )
    posm = comb[8:].astype(jnp.int32)
    validm = valid2[8:].astype(jnp.bfloat16)

    def placement(posall, valall, s):
        m = lax.broadcasted_iota(jnp.int32, (N_DEV, C), 0) == s
        prow = jnp.sum(posall * m, axis=0, keepdims=True)
        vrow = jnp.sum(valall * m.astype(jnp.bfloat16), axis=0, keepdims=True)
        eq = lax.broadcasted_iota(jnp.int32, (hc, C), 0) == prow
        return eq.astype(jnp.bfloat16) * vrow

    def body(b_ref, pos_ref, val_ref, out_ref, rbuf_ref, s1, r1, s2, r2):
        p = lax.axis_index("i")

        barrier_sem = pltpu.get_barrier_semaphore()
        for j in range(1, N_DEV):
            pl.semaphore_signal(
                barrier_sem, inc=1,
                device_id=(lax.bitwise_xor(p, j),),
                device_id_type=pl.DeviceIdType.MESH,
            )
        pl.semaphore_wait(barrier_sem, N_DEV - 1)

        sends1 = {}
        for j in _J_ORDER:
            q = lax.bitwise_xor(p, j)
            rdma = pltpu.make_async_remote_copy(
                src_ref=b_ref.at[pl.ds(_chunk_slot(q) * C, C)],
                dst_ref=rbuf_ref.at[j - 1],
                send_sem=s1.at[j - 1],
                recv_sem=r1.at[j - 1],
                device_id=(q,),
                device_id_type=pl.DeviceIdType.MESH,
            )
            rdma.start()
            sends1[j] = rdma

        tp = _chunk_slot(p)
        posall = pos_ref[...]
        valall = val_ref[...]
        acc = lax.dot_general(
            placement(posall, valall, p),
            b_ref[pl.ds(tp * C, C)],
            (((1,), (0,)), ((), ())),
            preferred_element_type=jnp.float32,
        )
        for j in _J_ORDER:
            sends1[j].wait_recv()
            s = lax.bitwise_xor(p, j)
            acc = acc + lax.dot_general(
                placement(posall, valall, s),
                rbuf_ref[j - 1],
                (((1,), (0,)), ((), ())),
                preferred_element_type=jnp.float32,
            )
        my_off = tp * hc
        out_ref[pl.ds(my_off, hc)] = acc.astype(jnp.bfloat16)

        sends2 = {}
        for j in _J_ORDER:
            q = lax.bitwise_xor(p, j)
            rdma = pltpu.make_async_remote_copy(
                src_ref=out_ref.at[pl.ds(my_off, hc)],
                dst_ref=out_ref.at[pl.ds(my_off, hc)],
                send_sem=s2.at[j - 1],
                recv_sem=r2.at[j - 1],
                device_id=(q,),
                device_id_type=pl.DeviceIdType.MESH,
            )
            rdma.start()
            sends2[j] = rdma
        for j in _J_ORDER:
            sends2[j].wait_recv()
        for j in _J_ORDER:
            sends1[j].wait_send()
            sends2[j].wait_send()

    out = pl.pallas_call(
        body,
        out_shape=jax.ShapeDtypeStruct((n, d), jnp.bfloat16),
        in_specs=[
            pl.BlockSpec(memory_space=pltpu.VMEM),
            pl.BlockSpec(memory_space=pltpu.VMEM),
            pl.BlockSpec(memory_space=pltpu.VMEM),
        ],
        out_specs=pl.BlockSpec(memory_space=pltpu.VMEM),
        scratch_shapes=[
            pltpu.VMEM((N_DEV - 1, C, d), jnp.bfloat16),
            pltpu.SemaphoreType.DMA((N_DEV - 1,)),
            pltpu.SemaphoreType.DMA((N_DEV - 1,)),
            pltpu.SemaphoreType.DMA((N_DEV - 1,)),
            pltpu.SemaphoreType.DMA((N_DEV - 1,)),
        ],
        compiler_params=pltpu.CompilerParams(collective_id=13),
    )(b, posm, validm)
    return out
